# baseline (speedup 1.0000x reference)
import functools

import jax
import jax.numpy as jnp
from jax import lax
from jax.experimental import pallas as pl
from jax.experimental.pallas import tpu as pltpu
from jax.experimental.pallas import tpu_sc as plsc

N = 10000
NH = N // 2
E = 320000
HID = 64
NCORE = 2
NSUB = 16
NW = NCORE * NSUB
NPAD = 10240
SLICE = NPAD // NSUB
C = 125
EPW = E // NW
NCH = EPW // C
DW = 16
RB = 1000
GRID = NH // RB

_mesh = plsc.VectorSubcoreMesh(core_axis_name="c", subcore_axis_name="s")
_sc_params = pltpu.CompilerParams(use_tc_tiling_on_sc=False)


def _sc_degree(edge3, ones_u, zrows):
    @functools.partial(
        pl.kernel,
        out_type=jax.ShapeDtypeStruct((NCORE, NPAD, HID), jnp.float32),
        mesh=_mesh,
        compiler_params=_sc_params,
        scratch_types=[
            pltpu.VMEM((NCH, C), jnp.int32),
            pltpu.VMEM((C, DW), jnp.float32),
            pltpu.VMEM_SHARED((NPAD, DW), jnp.float32),
        ],
    )
    def k(edge_hbm, ones_hbm, z_hbm, out_hbm, didx, ones_v, acc):
        c = lax.axis_index("c")
        s = lax.axis_index("s")
        w = c * NSUB + s
        pltpu.sync_copy(z_hbm, acc.at[pl.ds(s * SLICE, SLICE)])
        pltpu.sync_copy(edge_hbm.at[1, w], didx)
        pltpu.sync_copy(ones_hbm, ones_v)
        plsc.subcore_barrier()

        @pl.loop(0, NCH)
        def _(j):
            pltpu.sync_copy(ones_v, acc.at[didx.at[j]], add=True)

        plsc.subcore_barrier()
        for q in range(HID // DW):
            pltpu.sync_copy(acc.at[pl.ds(s * SLICE, SLICE)],
                            out_hbm.at[c, pl.ds(s * SLICE, SLICE),
                                       pl.ds(q * DW, DW)])

    return k(edge3, ones_u, zrows)


def _sc_scatter(hs_pk, edge3, zrows):
    @functools.partial(
        pl.kernel,
        out_type=jax.ShapeDtypeStruct((NCORE, NPAD, HID), jnp.float32),
        mesh=_mesh,
        compiler_params=_sc_params,
        scratch_types=[
            pltpu.VMEM((NCH, C), jnp.int32),
            pltpu.VMEM((NCH, C), jnp.int32),
            pltpu.VMEM((C, HID), jnp.float32),
            pltpu.VMEM((C, HID), jnp.float32),
            pltpu.SemaphoreType.DMA,
            pltpu.SemaphoreType.DMA,
            pltpu.VMEM_SHARED((NPAD, HID), jnp.float32),
        ],
    )
    def k(hs_hbm, edge_hbm, z_hbm, out_hbm, sidx, didx,
          rows_a, rows_b, sem_a, sem_b, acc):
        c = lax.axis_index("c")
        s = lax.axis_index("s")
        w = c * NSUB + s
        pltpu.sync_copy(z_hbm, acc.at[pl.ds(s * SLICE, SLICE)])
        pltpu.sync_copy(edge_hbm.at[0, w], sidx)
        pltpu.sync_copy(edge_hbm.at[1, w], didx)
        plsc.subcore_barrier()

        def fire(j, rows, sem):
            pltpu.async_copy(hs_hbm.at[sidx.at[j]], rows, sem)

        def drain(j, rows, sem):
            pltpu.make_async_copy(hs_hbm.at[sidx.at[j]], rows, sem).wait()

        fire(0, rows_a, sem_a)

        @pl.loop(0, NCH, step=2)
        def _(j):
            fire(j + 1, rows_b, sem_b)
            drain(j, rows_a, sem_a)
            pltpu.sync_copy(rows_a, acc.at[didx.at[j]], add=True)

            @pl.when(j + 2 < NCH)
            def _():
                fire(j + 2, rows_a, sem_a)

            drain(j + 1, rows_b, sem_b)
            pltpu.sync_copy(rows_b, acc.at[didx.at[j + 1]], add=True)

        plsc.subcore_barrier()
        pltpu.sync_copy(acc.at[pl.ds(s * SLICE, SLICE)],
                        out_hbm.at[c, pl.ds(s * SLICE, SLICE)])

    return k(hs_pk, edge3, zrows)


def _tc_mm(x, W):
    d = x.shape[1]

    def body(xa_ref, xb_ref, w_ref, o_ref):
        ha = jnp.dot(xa_ref[...], w_ref[...], preferred_element_type=jnp.float32)
        hb = jnp.dot(xb_ref[...], w_ref[...], preferred_element_type=jnp.float32)
        o_ref[...] = jnp.concatenate([ha, hb], axis=1)

    return pl.pallas_call(
        body,
        grid=(GRID,),
        in_specs=[
            pl.BlockSpec((RB, d), lambda i: (i, 0)),
            pl.BlockSpec((RB, d), lambda i: (i + GRID, 0)),
            pl.BlockSpec((d, HID), lambda i: (0, 0)),
        ],
        out_specs=pl.BlockSpec((RB, 2 * HID), lambda i: (i, 0)),
        out_shape=jax.ShapeDtypeStruct((NH, 2 * HID), jnp.float32),
    )(x, x, W)


def _tc_prescale(degp2, h2):
    def body(d_ref, h_ref, dinv_ref, hs_ref):
        dv = lax.rsqrt(d_ref[0] + d_ref[1] + 1.0)
        dinv_ref[...] = dv
        hs_ref[...] = h_ref[...] * dv

    return pl.pallas_call(
        body,
        grid=(GRID,),
        in_specs=[
            pl.BlockSpec((NCORE, RB, 2 * HID), lambda i: (0, i, 0)),
            pl.BlockSpec((RB, 2 * HID), lambda i: (i, 0)),
        ],
        out_specs=[
            pl.BlockSpec((RB, 2 * HID), lambda i: (i, 0)),
            pl.BlockSpec((RB, 2 * HID), lambda i: (i, 0)),
        ],
        out_shape=[
            jax.ShapeDtypeStruct((NH, 2 * HID), jnp.float32),
            jax.ShapeDtypeStruct((NH, 2 * HID), jnp.float32),
        ],
    )(degp2, h2)


def _half_tail(t, dv, b_ref, g_ref, bt_ref, L_ref, lb_ref):
    h = jnp.maximum(t * dv + b_ref[...], 0.0)
    mu = jnp.mean(h, axis=-1, keepdims=True)
    xc = h - mu
    var = jnp.mean(xc * xc, axis=-1, keepdims=True)
    hn = xc * lax.rsqrt(var + 1e-5) * g_ref[...] + bt_ref[...]
    h2 = jnp.dot(hn, L_ref[...], preferred_element_type=jnp.float32)
    return jnp.maximum(h2 + lb_ref[...], 0.0)


def _post_math(S_ref, hs_ref, dinv_ref, b_ref, g_ref, bt_ref, L_ref, lb_ref):
    t = S_ref[0] + S_ref[1] + hs_ref[...]
    dva = dinv_ref[...][:, 0:1]
    dvb = dinv_ref[...][:, HID:HID + 1]
    ha = _half_tail(t[:, :HID], dva, b_ref, g_ref, bt_ref, L_ref, lb_ref)
    hb = _half_tail(t[:, HID:], dvb, b_ref, g_ref, bt_ref, L_ref, lb_ref)
    return ha, hb, dva, dvb


def _tc_post(S2, hs2, dinv2, b, g, bt, L, lb, Wn):
    d2 = L.shape[1]

    def body(S_ref, hs_ref, dinv_ref, b_ref, g_ref, bt_ref, L_ref, lb_ref,
             wn_ref, o_ref):
        ha, hb, dva, dvb = _post_math(S_ref, hs_ref, dinv_ref, b_ref, g_ref,
                                      bt_ref, L_ref, lb_ref)
        oa = jnp.dot(ha, wn_ref[...], preferred_element_type=jnp.float32) * dva
        ob = jnp.dot(hb, wn_ref[...], preferred_element_type=jnp.float32) * dvb
        o_ref[...] = jnp.concatenate([oa, ob], axis=1)

    return pl.pallas_call(
        body,
        grid=(GRID,),
        in_specs=[
            pl.BlockSpec((NCORE, RB, 2 * HID), lambda i: (0, i, 0)),
            pl.BlockSpec((RB, 2 * HID), lambda i: (i, 0)),
            pl.BlockSpec((RB, 2 * HID), lambda i: (i, 0)),
            pl.BlockSpec((1, HID), lambda i: (0, 0)),
            pl.BlockSpec((1, HID), lambda i: (0, 0)),
            pl.BlockSpec((1, HID), lambda i: (0, 0)),
            pl.BlockSpec((HID, d2), lambda i: (0, 0)),
            pl.BlockSpec((1, d2), lambda i: (0, 0)),
            pl.BlockSpec((d2, HID), lambda i: (0, 0)),
        ],
        out_specs=pl.BlockSpec((RB, 2 * HID), lambda i: (i, 0)),
        out_shape=jax.ShapeDtypeStruct((NH, 2 * HID), jnp.float32),
    )(S2, hs2, dinv2, b, g, bt, L, lb, Wn)


def _tc_final(S2, hs2, dinv2, b, g, bt, L, lb):
    d2 = L.shape[1]

    def body(S_ref, hs_ref, dinv_ref, b_ref, g_ref, bt_ref, L_ref, lb_ref,
             o_ref):
        ha, hb, _, _ = _post_math(S_ref, hs_ref, dinv_ref, b_ref, g_ref,
                                  bt_ref, L_ref, lb_ref)
        m = jnp.maximum(jnp.max(ha, axis=0, keepdims=True),
                        jnp.max(hb, axis=0, keepdims=True))
        i = pl.program_id(0)

        @pl.when(i == 0)
        def _():
            o_ref[...] = m

        @pl.when(i > 0)
        def _():
            o_ref[...] = jnp.maximum(o_ref[...], m)

    return pl.pallas_call(
        body,
        grid=(GRID,),
        in_specs=[
            pl.BlockSpec((NCORE, RB, 2 * HID), lambda i: (0, i, 0)),
            pl.BlockSpec((RB, 2 * HID), lambda i: (i, 0)),
            pl.BlockSpec((RB, 2 * HID), lambda i: (i, 0)),
            pl.BlockSpec((1, HID), lambda i: (0, 0)),
            pl.BlockSpec((1, HID), lambda i: (0, 0)),
            pl.BlockSpec((1, HID), lambda i: (0, 0)),
            pl.BlockSpec((HID, d2), lambda i: (0, 0)),
            pl.BlockSpec((1, d2), lambda i: (0, 0)),
        ],
        out_specs=pl.BlockSpec((1, d2), lambda i: (0, 0)),
        out_shape=jax.ShapeDtypeStruct((1, d2), jnp.float32),
    )(S2, hs2, dinv2, b, g, bt, L, lb)


def kernel(x, edge_index,
           W1, b1, g1, bt1, L1, lb1,
           W2, b2, g2, bt2, L2, lb2,
           W3, b3, g3, bt3, L3, lb3):
    epi = (edge_index % NH) * 2 + edge_index // NH
    edge3 = epi.reshape(2, NW, NCH, C)

    z_deg = jnp.zeros((SLICE, DW), jnp.float32)
    z_acc = jnp.zeros((SLICE, HID), jnp.float32)
    ones_u = jnp.ones((C, DW), jnp.float32)

    row = lambda v: v.reshape(1, -1)
    pack2 = lambda a: a.reshape(NCORE, NPAD // 2, 2 * HID)

    degp = _sc_degree(edge3, ones_u, z_deg)
    h2 = _tc_mm(x, W1)
    dinv2, hs2 = _tc_prescale(pack2(degp), h2)

    unpack = lambda a: a.reshape(N, HID)
    S = pack2(_sc_scatter(unpack(hs2), edge3, z_acc))
    hs2 = _tc_post(S, hs2, dinv2, row(b1), row(g1), row(bt1), L1, row(lb1), W2)

    S = pack2(_sc_scatter(unpack(hs2), edge3, z_acc))
    hs2 = _tc_post(S, hs2, dinv2, row(b2), row(g2), row(bt2), L2, row(lb2), W3)

    S = pack2(_sc_scatter(unpack(hs2), edge3, z_acc))
    out = _tc_final(S, hs2, dinv2, row(b3), row(g3), row(bt3), L3, row(lb3))
    return out.reshape(L3.shape[1])


# speedup vs baseline: 1.0903x; 1.0903x over previous
import functools

import jax
import jax.numpy as jnp
from jax import lax
from jax.experimental import pallas as pl
from jax.experimental.pallas import tpu as pltpu
from jax.experimental.pallas import tpu_sc as plsc

N = 10000
NH = N // 2
E = 320000
HID = 64
NCORE = 2
NSUB = 16
NW = NCORE * NSUB
NPAD = 10240
SLICE = NPAD // NSUB
C = 125
EPW = E // NW
NCH = EPW // C
DW = 16
RB = 1000
GRID = NH // RB

_mesh = plsc.VectorSubcoreMesh(core_axis_name="c", subcore_axis_name="s")
_sc_params = pltpu.CompilerParams(use_tc_tiling_on_sc=False)


def _sc_degree(edge3, ones_u, zrows):
    @functools.partial(
        pl.kernel,
        out_type=jax.ShapeDtypeStruct((NCORE, NPAD, DW), jnp.float32),
        mesh=_mesh,
        compiler_params=_sc_params,
        scratch_types=[
            pltpu.VMEM((NCH, C), jnp.int32),
            pltpu.VMEM((C, DW), jnp.float32),
            pltpu.VMEM_SHARED((NPAD, DW), jnp.float32),
        ],
    )
    def k(edge_hbm, ones_hbm, z_hbm, out_hbm, didx, ones_v, acc):
        c = lax.axis_index("c")
        s = lax.axis_index("s")
        w = c * NSUB + s
        pltpu.sync_copy(z_hbm, acc.at[pl.ds(s * SLICE, SLICE)])
        pltpu.sync_copy(edge_hbm.at[1, w], didx)
        pltpu.sync_copy(ones_hbm, ones_v)
        plsc.subcore_barrier()

        @pl.loop(0, NCH)
        def _(j):
            pltpu.sync_copy(ones_v, acc.at[didx.at[j]], add=True)

        plsc.subcore_barrier()
        pltpu.sync_copy(acc.at[pl.ds(s * SLICE, SLICE)],
                        out_hbm.at[c, pl.ds(s * SLICE, SLICE)])

    return k(edge3, ones_u, zrows)


def _sc_scatter(hs_pk, edge3, zrows):
    @functools.partial(
        pl.kernel,
        out_type=jax.ShapeDtypeStruct((NCORE, NPAD, HID), jnp.float32),
        mesh=_mesh,
        compiler_params=_sc_params,
        scratch_types=[
            pltpu.VMEM((NCH, C), jnp.int32),
            pltpu.VMEM((NCH, C), jnp.int32),
            pltpu.VMEM((C, HID), jnp.float32),
            pltpu.VMEM((C, HID), jnp.float32),
            pltpu.SemaphoreType.DMA,
            pltpu.SemaphoreType.DMA,
            pltpu.VMEM_SHARED((NPAD, HID), jnp.float32),
        ],
    )
    def k(hs_hbm, edge_hbm, z_hbm, out_hbm, sidx, didx,
          rows_a, rows_b, sem_a, sem_b, acc):
        c = lax.axis_index("c")
        s = lax.axis_index("s")
        w = c * NSUB + s
        pltpu.sync_copy(z_hbm, acc.at[pl.ds(s * SLICE, SLICE)])
        pltpu.sync_copy(edge_hbm.at[0, w], sidx)
        pltpu.sync_copy(edge_hbm.at[1, w], didx)
        plsc.subcore_barrier()

        def fire(j, rows, sem):
            pltpu.async_copy(hs_hbm.at[sidx.at[j]], rows, sem)

        def drain(j, rows, sem):
            pltpu.make_async_copy(hs_hbm.at[sidx.at[j]], rows, sem).wait()

        fire(0, rows_a, sem_a)

        @pl.loop(0, NCH, step=2)
        def _(j):
            fire(j + 1, rows_b, sem_b)
            drain(j, rows_a, sem_a)
            pltpu.sync_copy(rows_a, acc.at[didx.at[j]], add=True)

            @pl.when(j + 2 < NCH)
            def _():
                fire(j + 2, rows_a, sem_a)

            drain(j + 1, rows_b, sem_b)
            pltpu.sync_copy(rows_b, acc.at[didx.at[j + 1]], add=True)

        plsc.subcore_barrier()
        pltpu.sync_copy(acc.at[pl.ds(s * SLICE, SLICE)],
                        out_hbm.at[c, pl.ds(s * SLICE, SLICE)])

    return k(hs_pk, edge3, zrows)


def _tc_mm(x, W):
    d = x.shape[1]

    def body(xa_ref, xb_ref, w_ref, o_ref):
        ha = jnp.dot(xa_ref[...], w_ref[...], preferred_element_type=jnp.float32)
        hb = jnp.dot(xb_ref[...], w_ref[...], preferred_element_type=jnp.float32)
        o_ref[...] = jnp.concatenate([ha, hb], axis=1)

    return pl.pallas_call(
        body,
        grid=(GRID,),
        in_specs=[
            pl.BlockSpec((RB, d), lambda i: (i, 0)),
            pl.BlockSpec((RB, d), lambda i: (i + GRID, 0)),
            pl.BlockSpec((d, HID), lambda i: (0, 0)),
        ],
        out_specs=pl.BlockSpec((RB, 2 * HID), lambda i: (i, 0)),
        out_shape=jax.ShapeDtypeStruct((NH, 2 * HID), jnp.float32),
    )(x, x, W)


def _tc_prescale(degp2, h2):
    def body(d_ref, h_ref, dinv_ref, hs_ref):
        dv = lax.rsqrt(d_ref[0] + d_ref[1] + 1.0)
        dinv_ref[...] = dv
        hs_ref[...] = h_ref[...] * dv

    return pl.pallas_call(
        body,
        grid=(GRID,),
        in_specs=[
            pl.BlockSpec((NCORE, RB, 2 * HID), lambda i: (0, i, 0)),
            pl.BlockSpec((RB, 2 * HID), lambda i: (i, 0)),
        ],
        out_specs=[
            pl.BlockSpec((RB, 2 * HID), lambda i: (i, 0)),
            pl.BlockSpec((RB, 2 * HID), lambda i: (i, 0)),
        ],
        out_shape=[
            jax.ShapeDtypeStruct((NH, 2 * HID), jnp.float32),
            jax.ShapeDtypeStruct((NH, 2 * HID), jnp.float32),
        ],
    )(degp2, h2)


def _post_math(S_ref, hs_ref, dinv_ref, b_ref, g_ref, bt_ref, LL_ref,
               lb_ref, P_ref):
    t = S_ref[0] + S_ref[1] + hs_ref[...]
    h = jnp.maximum(t * dinv_ref[...] + b_ref[...], 0.0)
    mu = jnp.dot(h, P_ref[...], preferred_element_type=jnp.float32)
    xc = h - mu
    var = jnp.dot(xc * xc, P_ref[...], preferred_element_type=jnp.float32)
    hn = xc * lax.rsqrt(var + 1e-5) * g_ref[...] + bt_ref[...]
    h2 = jnp.dot(hn, LL_ref[...], preferred_element_type=jnp.float32)
    return jnp.maximum(h2 + lb_ref[...], 0.0)


def _tc_post(S2, hs2, dinv2, b2, g2, bt2, LL, lb2, WW, P):
    dd = LL.shape[1]

    def body(S_ref, hs_ref, dinv_ref, b_ref, g_ref, bt_ref, LL_ref, lb_ref,
             ww_ref, p_ref, o_ref):
        h2 = _post_math(S_ref, hs_ref, dinv_ref, b_ref, g_ref, bt_ref,
                        LL_ref, lb_ref, p_ref)
        o_ref[...] = jnp.dot(h2, ww_ref[...],
                             preferred_element_type=jnp.float32) * dinv_ref[...]

    return pl.pallas_call(
        body,
        grid=(GRID,),
        in_specs=[
            pl.BlockSpec((NCORE, RB, 2 * HID), lambda i: (0, i, 0)),
            pl.BlockSpec((RB, 2 * HID), lambda i: (i, 0)),
            pl.BlockSpec((RB, 2 * HID), lambda i: (i, 0)),
            pl.BlockSpec((1, 2 * HID), lambda i: (0, 0)),
            pl.BlockSpec((1, 2 * HID), lambda i: (0, 0)),
            pl.BlockSpec((1, 2 * HID), lambda i: (0, 0)),
            pl.BlockSpec((2 * HID, dd), lambda i: (0, 0)),
            pl.BlockSpec((1, dd), lambda i: (0, 0)),
            pl.BlockSpec((dd, 2 * HID), lambda i: (0, 0)),
            pl.BlockSpec((2 * HID, 2 * HID), lambda i: (0, 0)),
        ],
        out_specs=pl.BlockSpec((RB, 2 * HID), lambda i: (i, 0)),
        out_shape=jax.ShapeDtypeStruct((NH, 2 * HID), jnp.float32),
    )(S2, hs2, dinv2, b2, g2, bt2, LL, lb2, WW, P)


def _tc_final(S2, hs2, dinv2, b2, g2, bt2, LL, lb2, P):
    dd = LL.shape[1]
    d2 = dd // 2

    def body(S_ref, hs_ref, dinv_ref, b_ref, g_ref, bt_ref, LL_ref, lb_ref,
             p_ref, o_ref):
        h2 = _post_math(S_ref, hs_ref, dinv_ref, b_ref, g_ref, bt_ref,
                        LL_ref, lb_ref, p_ref)
        m2 = jnp.max(h2, axis=0, keepdims=True)
        m = jnp.maximum(m2[:, :d2], m2[:, d2:])
        i = pl.program_id(0)

        @pl.when(i == 0)
        def _():
            o_ref[...] = m

        @pl.when(i > 0)
        def _():
            o_ref[...] = jnp.maximum(o_ref[...], m)

    return pl.pallas_call(
        body,
        grid=(GRID,),
        in_specs=[
            pl.BlockSpec((NCORE, RB, 2 * HID), lambda i: (0, i, 0)),
            pl.BlockSpec((RB, 2 * HID), lambda i: (i, 0)),
            pl.BlockSpec((RB, 2 * HID), lambda i: (i, 0)),
            pl.BlockSpec((1, 2 * HID), lambda i: (0, 0)),
            pl.BlockSpec((1, 2 * HID), lambda i: (0, 0)),
            pl.BlockSpec((1, 2 * HID), lambda i: (0, 0)),
            pl.BlockSpec((2 * HID, dd), lambda i: (0, 0)),
            pl.BlockSpec((1, dd), lambda i: (0, 0)),
            pl.BlockSpec((2 * HID, 2 * HID), lambda i: (0, 0)),
        ],
        out_specs=pl.BlockSpec((1, d2), lambda i: (0, 0)),
        out_shape=jax.ShapeDtypeStruct((1, d2), jnp.float32),
    )(S2, hs2, dinv2, b2, g2, bt2, LL, lb2, P)


def kernel(x, edge_index,
           W1, b1, g1, bt1, L1, lb1,
           W2, b2, g2, bt2, L2, lb2,
           W3, b3, g3, bt3, L3, lb3):
    epi = (edge_index % NH) * 2 + edge_index // NH
    edge3 = epi.reshape(2, NW, NCH, C)

    z_deg = jnp.zeros((SLICE, DW), jnp.float32)
    z_acc = jnp.zeros((SLICE, HID), jnp.float32)
    ones_u = jnp.ones((C, DW), jnp.float32)

    def dup(v):
        return jnp.concatenate([v, v]).reshape(1, -1)

    def bdiag(M):
        a, b = M.shape
        Z = jnp.zeros((a, b), M.dtype)
        return jnp.concatenate(
            [jnp.concatenate([M, Z], axis=1),
             jnp.concatenate([Z, M], axis=1)], axis=0)

    P = bdiag(jnp.full((HID, HID), 1.0 / HID, jnp.float32))
    pack2 = lambda a: a.reshape(NCORE, NPAD // 2, 2 * HID)
    unpack = lambda a: a.reshape(N, HID)

    degp = _sc_degree(edge3, ones_u, z_deg)
    h2 = _tc_mm(x, W1)
    degp2 = pack2(jnp.tile(degp, (1, 1, HID // DW)))
    dinv2, hs2 = _tc_prescale(degp2, h2)

    S = pack2(_sc_scatter(unpack(hs2), edge3, z_acc))
    hs2 = _tc_post(S, hs2, dinv2, dup(b1), dup(g1), dup(bt1), bdiag(L1),
                   dup(lb1), bdiag(W2), P)

    S = pack2(_sc_scatter(unpack(hs2), edge3, z_acc))
    hs2 = _tc_post(S, hs2, dinv2, dup(b2), dup(g2), dup(bt2), bdiag(L2),
                   dup(lb2), bdiag(W3), P)

    S = pack2(_sc_scatter(unpack(hs2), edge3, z_acc))
    out = _tc_final(S, hs2, dinv2, dup(b3), dup(g3), dup(bt3), bdiag(L3),
                    dup(lb3), P)
    return out.reshape(L3.shape[1])


# speedup vs baseline: 1.2785x; 1.1726x over previous
import functools

import jax
import jax.numpy as jnp
from jax import lax
from jax.experimental import pallas as pl
from jax.experimental.pallas import tpu as pltpu
from jax.experimental.pallas import tpu_sc as plsc

N = 10000
NH = N // 2
E = 320000
HID = 64
NCORE = 2
NSUB = 16
NW = NCORE * NSUB
NPAD = 10240
SLICE = NPAD // NSUB
C = 125
EPW = E // NW
NCH = EPW // C
DW = 16
RB = 1000
GRID = NH // RB

_mesh = plsc.VectorSubcoreMesh(core_axis_name="c", subcore_axis_name="s")
_sc_params = pltpu.CompilerParams(use_tc_tiling_on_sc=False)


def _sc_degree(edge3, ones_u, zrows):
    @functools.partial(
        pl.kernel,
        out_type=jax.ShapeDtypeStruct((NCORE, NPAD, HID // DW, DW), jnp.float32),
        mesh=_mesh,
        compiler_params=_sc_params,
        scratch_types=[
            pltpu.VMEM((NCH, C), jnp.int32),
            pltpu.VMEM((C, DW), jnp.float32),
            pltpu.VMEM((SLICE, DW), jnp.float32),
            pltpu.VMEM((SLICE, HID // DW, DW), jnp.float32),
            pltpu.VMEM_SHARED((NPAD, DW), jnp.float32),
        ],
    )
    def k(edge_hbm, ones_hbm, z_hbm, out_hbm, didx, ones_v, nar, wide, acc):
        c = lax.axis_index("c")
        s = lax.axis_index("s")
        w = c * NSUB + s
        pltpu.sync_copy(z_hbm, acc.at[pl.ds(s * SLICE, SLICE)])
        pltpu.sync_copy(edge_hbm.at[1, w], didx)
        pltpu.sync_copy(ones_hbm, ones_v)
        plsc.subcore_barrier()

        @pl.loop(0, NCH)
        def _(j):
            pltpu.sync_copy(ones_v, acc.at[didx.at[j]], add=True)

        plsc.subcore_barrier()
        pltpu.sync_copy(acc.at[pl.ds(s * SLICE, SLICE)], nar)

        @pl.loop(0, SLICE)
        def _(r):
            v = nar[r]
            for q in range(HID // DW):
                wide[r, q] = v

        pltpu.sync_copy(wide, out_hbm.at[c, pl.ds(s * SLICE, SLICE)])

    return k(edge3, ones_u, zrows)


def _sc_scatter(hs_pk, edge3, zrows):
    @functools.partial(
        pl.kernel,
        out_type=jax.ShapeDtypeStruct((NCORE, NPAD, HID), jnp.float32),
        mesh=_mesh,
        compiler_params=_sc_params,
        scratch_types=[
            pltpu.VMEM((NCH, C), jnp.int32),
            pltpu.VMEM((NCH, C), jnp.int32),
            pltpu.VMEM((C, HID), jnp.float32),
            pltpu.VMEM((C, HID), jnp.float32),
            pltpu.SemaphoreType.DMA,
            pltpu.SemaphoreType.DMA,
            pltpu.VMEM_SHARED((NPAD, HID), jnp.float32),
        ],
    )
    def k(hs_hbm, edge_hbm, z_hbm, out_hbm, sidx, didx,
          rows_a, rows_b, sem_a, sem_b, acc):
        c = lax.axis_index("c")
        s = lax.axis_index("s")
        w = c * NSUB + s
        pltpu.sync_copy(z_hbm, acc.at[pl.ds(s * SLICE, SLICE)])
        pltpu.sync_copy(edge_hbm.at[0, w], sidx)
        pltpu.sync_copy(edge_hbm.at[1, w], didx)
        plsc.subcore_barrier()

        def fire(j, rows, sem):
            pltpu.async_copy(hs_hbm.at[sidx.at[j]], rows, sem)

        def drain(j, rows, sem):
            pltpu.make_async_copy(hs_hbm.at[sidx.at[j]], rows, sem).wait()

        fire(0, rows_a, sem_a)

        @pl.loop(0, NCH, step=2)
        def _(j):
            fire(j + 1, rows_b, sem_b)
            drain(j, rows_a, sem_a)
            pltpu.sync_copy(rows_a, acc.at[didx.at[j]], add=True)

            @pl.when(j + 2 < NCH)
            def _():
                fire(j + 2, rows_a, sem_a)

            drain(j + 1, rows_b, sem_b)
            pltpu.sync_copy(rows_b, acc.at[didx.at[j + 1]], add=True)

        plsc.subcore_barrier()
        pltpu.sync_copy(acc.at[pl.ds(s * SLICE, SLICE)],
                        out_hbm.at[c, pl.ds(s * SLICE, SLICE)])

    return k(hs_pk, edge3, zrows)


def _tc_mm(x, W):
    d = x.shape[1]

    def body(xa_ref, xb_ref, w_ref, o_ref):
        ha = jnp.dot(xa_ref[...], w_ref[...], preferred_element_type=jnp.float32)
        hb = jnp.dot(xb_ref[...], w_ref[...], preferred_element_type=jnp.float32)
        o_ref[...] = jnp.concatenate([ha, hb], axis=1)

    return pl.pallas_call(
        body,
        grid=(GRID,),
        in_specs=[
            pl.BlockSpec((RB, d), lambda i: (i, 0)),
            pl.BlockSpec((RB, d), lambda i: (i + GRID, 0)),
            pl.BlockSpec((d, HID), lambda i: (0, 0)),
        ],
        out_specs=pl.BlockSpec((RB, 2 * HID), lambda i: (i, 0)),
        out_shape=jax.ShapeDtypeStruct((NH, 2 * HID), jnp.float32),
    )(x, x, W)


def _tc_prescale(degp2, h2):
    def body(d_ref, h_ref, dinv_ref, hs_ref):
        dv = lax.rsqrt(d_ref[0] + d_ref[1] + 1.0)
        dinv_ref[...] = dv
        hs_ref[...] = h_ref[...] * dv

    return pl.pallas_call(
        body,
        grid=(GRID,),
        in_specs=[
            pl.BlockSpec((NCORE, RB, 2 * HID), lambda i: (0, i, 0)),
            pl.BlockSpec((RB, 2 * HID), lambda i: (i, 0)),
        ],
        out_specs=[
            pl.BlockSpec((RB, 2 * HID), lambda i: (i, 0)),
            pl.BlockSpec((RB, 2 * HID), lambda i: (i, 0)),
        ],
        out_shape=[
            jax.ShapeDtypeStruct((NH, 2 * HID), jnp.float32),
            jax.ShapeDtypeStruct((NH, 2 * HID), jnp.float32),
        ],
    )(degp2, h2)


def _post_math(S_ref, hs_ref, dinv_ref, b_ref, g_ref, bt_ref, LL_ref,
               lb_ref, P_ref):
    t = S_ref[0] + S_ref[1] + hs_ref[...]
    h = jnp.maximum(t * dinv_ref[...] + b_ref[...], 0.0)
    mu = jnp.dot(h, P_ref[...], preferred_element_type=jnp.float32)
    xc = h - mu
    var = jnp.dot(xc * xc, P_ref[...], preferred_element_type=jnp.float32)
    hn = xc * lax.rsqrt(var + 1e-5) * g_ref[...] + bt_ref[...]
    h2 = jnp.dot(hn, LL_ref[...], preferred_element_type=jnp.float32)
    return jnp.maximum(h2 + lb_ref[...], 0.0)


def _tc_post(S2, hs2, dinv2, b2, g2, bt2, LL, lb2, WW, P):
    dd = LL.shape[1]

    def body(S_ref, hs_ref, dinv_ref, b_ref, g_ref, bt_ref, LL_ref, lb_ref,
             ww_ref, p_ref, o_ref):
        h2 = _post_math(S_ref, hs_ref, dinv_ref, b_ref, g_ref, bt_ref,
                        LL_ref, lb_ref, p_ref)
        o_ref[...] = jnp.dot(h2, ww_ref[...],
                             preferred_element_type=jnp.float32) * dinv_ref[...]

    return pl.pallas_call(
        body,
        grid=(GRID,),
        in_specs=[
            pl.BlockSpec((NCORE, RB, 2 * HID), lambda i: (0, i, 0)),
            pl.BlockSpec((RB, 2 * HID), lambda i: (i, 0)),
            pl.BlockSpec((RB, 2 * HID), lambda i: (i, 0)),
            pl.BlockSpec((1, 2 * HID), lambda i: (0, 0)),
            pl.BlockSpec((1, 2 * HID), lambda i: (0, 0)),
            pl.BlockSpec((1, 2 * HID), lambda i: (0, 0)),
            pl.BlockSpec((2 * HID, dd), lambda i: (0, 0)),
            pl.BlockSpec((1, dd), lambda i: (0, 0)),
            pl.BlockSpec((dd, 2 * HID), lambda i: (0, 0)),
            pl.BlockSpec((2 * HID, 2 * HID), lambda i: (0, 0)),
        ],
        out_specs=pl.BlockSpec((RB, 2 * HID), lambda i: (i, 0)),
        out_shape=jax.ShapeDtypeStruct((NH, 2 * HID), jnp.float32),
    )(S2, hs2, dinv2, b2, g2, bt2, LL, lb2, WW, P)


def _tc_final(S2, hs2, dinv2, b2, g2, bt2, LL, lb2, P):
    dd = LL.shape[1]
    d2 = dd // 2

    def body(S_ref, hs_ref, dinv_ref, b_ref, g_ref, bt_ref, LL_ref, lb_ref,
             p_ref, o_ref):
        h2 = _post_math(S_ref, hs_ref, dinv_ref, b_ref, g_ref, bt_ref,
                        LL_ref, lb_ref, p_ref)
        m2 = jnp.max(h2, axis=0, keepdims=True)
        m = jnp.maximum(m2[:, :d2], m2[:, d2:])
        i = pl.program_id(0)

        @pl.when(i == 0)
        def _():
            o_ref[...] = m

        @pl.when(i > 0)
        def _():
            o_ref[...] = jnp.maximum(o_ref[...], m)

    return pl.pallas_call(
        body,
        grid=(GRID,),
        in_specs=[
            pl.BlockSpec((NCORE, RB, 2 * HID), lambda i: (0, i, 0)),
            pl.BlockSpec((RB, 2 * HID), lambda i: (i, 0)),
            pl.BlockSpec((RB, 2 * HID), lambda i: (i, 0)),
            pl.BlockSpec((1, 2 * HID), lambda i: (0, 0)),
            pl.BlockSpec((1, 2 * HID), lambda i: (0, 0)),
            pl.BlockSpec((1, 2 * HID), lambda i: (0, 0)),
            pl.BlockSpec((2 * HID, dd), lambda i: (0, 0)),
            pl.BlockSpec((1, dd), lambda i: (0, 0)),
            pl.BlockSpec((2 * HID, 2 * HID), lambda i: (0, 0)),
        ],
        out_specs=pl.BlockSpec((1, d2), lambda i: (0, 0)),
        out_shape=jax.ShapeDtypeStruct((1, d2), jnp.float32),
    )(S2, hs2, dinv2, b2, g2, bt2, LL, lb2, P)


def kernel(x, edge_index,
           W1, b1, g1, bt1, L1, lb1,
           W2, b2, g2, bt2, L2, lb2,
           W3, b3, g3, bt3, L3, lb3):
    epi = (edge_index % NH) * 2 + edge_index // NH
    edge3 = epi.reshape(2, NW, NCH, C)

    z_deg = jnp.zeros((SLICE, DW), jnp.float32)
    z_acc = jnp.zeros((SLICE, HID), jnp.float32)
    ones_u = jnp.ones((C, DW), jnp.float32)

    def dup(v):
        return jnp.concatenate([v, v]).reshape(1, -1)

    def bdiag(M):
        a, b = M.shape
        Z = jnp.zeros((a, b), M.dtype)
        return jnp.concatenate(
            [jnp.concatenate([M, Z], axis=1),
             jnp.concatenate([Z, M], axis=1)], axis=0)

    P = bdiag(jnp.full((HID, HID), 1.0 / HID, jnp.float32))
    pack2 = lambda a: a.reshape(NCORE, NPAD // 2, 2 * HID)
    unpack = lambda a: a.reshape(N, HID)

    degp = _sc_degree(edge3, ones_u, z_deg)
    h2 = _tc_mm(x, W1)
    degp2 = degp.reshape(NCORE, NPAD // 2, 2 * HID)
    dinv2, hs2 = _tc_prescale(degp2, h2)

    S = pack2(_sc_scatter(unpack(hs2), edge3, z_acc))
    hs2 = _tc_post(S, hs2, dinv2, dup(b1), dup(g1), dup(bt1), bdiag(L1),
                   dup(lb1), bdiag(W2), P)

    S = pack2(_sc_scatter(unpack(hs2), edge3, z_acc))
    hs2 = _tc_post(S, hs2, dinv2, dup(b2), dup(g2), dup(bt2), bdiag(L2),
                   dup(lb2), bdiag(W3), P)

    S = pack2(_sc_scatter(unpack(hs2), edge3, z_acc))
    out = _tc_final(S, hs2, dinv2, dup(b3), dup(g3), dup(bt3), bdiag(L3),
                    dup(lb3), P)
    return out.reshape(L3.shape[1])


# speedup vs baseline: 1.3678x; 1.0698x over previous
import functools

import jax
import jax.numpy as jnp
from jax import lax
from jax.experimental import pallas as pl
from jax.experimental.pallas import tpu as pltpu
from jax.experimental.pallas import tpu_sc as plsc

N = 10000
NH = N // 2
E = 320000
HID = 64
NCORE = 2
NSUB = 16
NW = NCORE * NSUB
NPAD = 10240
SLICE = NPAD // NSUB
C = 125
EPW = E // NW
NCH = EPW // C
DW = 16
RB = 1000
GRID = NH // RB

_mesh = plsc.VectorSubcoreMesh(core_axis_name="c", subcore_axis_name="s")
_sc_params = pltpu.CompilerParams(use_tc_tiling_on_sc=False)


def _sc_degree(edge3, ones_u, zrows):
    @functools.partial(
        pl.kernel,
        out_type=jax.ShapeDtypeStruct((NCORE, NPAD, HID // DW, DW), jnp.float32),
        mesh=_mesh,
        compiler_params=_sc_params,
        scratch_types=[
            pltpu.VMEM((NCH, C), jnp.int32),
            pltpu.VMEM((C, DW), jnp.float32),
            pltpu.VMEM((SLICE, DW), jnp.float32),
            pltpu.VMEM((SLICE, HID // DW, DW), jnp.float32),
            pltpu.VMEM_SHARED((NPAD, DW), jnp.float32),
        ],
    )
    def k(edge_hbm, ones_hbm, z_hbm, out_hbm, didx, ones_v, nar, wide, acc):
        c = lax.axis_index("c")
        s = lax.axis_index("s")
        w = c * NSUB + s
        pltpu.sync_copy(z_hbm, acc.at[pl.ds(s * SLICE, SLICE)])
        pltpu.sync_copy(edge_hbm.at[1, w], didx)
        pltpu.sync_copy(ones_hbm, ones_v)
        plsc.subcore_barrier()

        @pl.loop(0, NCH)
        def _(j):
            pltpu.sync_copy(ones_v, acc.at[didx.at[j]], add=True)

        plsc.subcore_barrier()
        pltpu.sync_copy(acc.at[pl.ds(s * SLICE, SLICE)], nar)

        @pl.loop(0, SLICE)
        def _(r):
            v = nar[r]
            for q in range(HID // DW):
                wide[r, q] = v

        pltpu.sync_copy(wide, out_hbm.at[c, pl.ds(s * SLICE, SLICE)])

    return k(edge3, ones_u, zrows)


def _sc_scatter(hs_pk, edge3, zrows):
    @functools.partial(
        pl.kernel,
        out_type=jax.ShapeDtypeStruct((NCORE, NPAD, HID), jnp.float32),
        mesh=_mesh,
        compiler_params=_sc_params,
        scratch_types=[
            pltpu.VMEM((NCH, C), jnp.int32),
            pltpu.VMEM((NCH, C), jnp.int32),
            [pltpu.VMEM((C, HID), jnp.float32)] * 4,
            [pltpu.SemaphoreType.DMA] * 4,
            [pltpu.SemaphoreType.DMA] * 4,
            pltpu.VMEM_SHARED((NPAD, HID), jnp.float32),
        ],
    )
    def k(hs_hbm, edge_hbm, z_hbm, out_hbm, sidx, didx,
          rows, gsem, ssem, acc):
        c = lax.axis_index("c")
        s = lax.axis_index("s")
        w = c * NSUB + s
        pltpu.sync_copy(z_hbm, acc.at[pl.ds(s * SLICE, SLICE)])
        pltpu.sync_copy(edge_hbm.at[0, w], sidx)
        pltpu.sync_copy(edge_hbm.at[1, w], didx)
        plsc.subcore_barrier()

        def fire_g(j, q):
            pltpu.async_copy(hs_hbm.at[sidx.at[j]], rows[q], gsem[q])

        def drain_g(j, q):
            pltpu.make_async_copy(hs_hbm.at[sidx.at[j]], rows[q],
                                  gsem[q]).wait()

        def fire_s(j, q):
            pltpu.async_copy(rows[q], acc.at[didx.at[j]], ssem[q], add=True)

        def drain_s(j, q):
            pltpu.make_async_copy(rows[q], acc.at[didx.at[j]],
                                  ssem[q]).wait()

        for q in range(4):
            fire_g(q, q)

        @pl.loop(0, NCH, step=4)
        def _(j):
            for q in range(4):
                drain_g(j + q, q)
                fire_s(j + q, q)

            @pl.when(j + 4 < NCH)
            def _():
                for q in range(4):
                    drain_s(j + q, q)
                    fire_g(j + 4 + q, q)

        for q in range(4):
            drain_s(NCH - 4 + q, q)

        plsc.subcore_barrier()
        pltpu.sync_copy(acc.at[pl.ds(s * SLICE, SLICE)],
                        out_hbm.at[c, pl.ds(s * SLICE, SLICE)])

    return k(hs_pk, edge3, zrows)


def _tc_mm(x, W):
    d = x.shape[1]

    def body(xa_ref, xb_ref, w_ref, o_ref):
        ha = jnp.dot(xa_ref[...], w_ref[...], preferred_element_type=jnp.float32)
        hb = jnp.dot(xb_ref[...], w_ref[...], preferred_element_type=jnp.float32)
        o_ref[...] = jnp.concatenate([ha, hb], axis=1)

    return pl.pallas_call(
        body,
        grid=(GRID,),
        in_specs=[
            pl.BlockSpec((RB, d), lambda i: (i, 0)),
            pl.BlockSpec((RB, d), lambda i: (i + GRID, 0)),
            pl.BlockSpec((d, HID), lambda i: (0, 0)),
        ],
        out_specs=pl.BlockSpec((RB, 2 * HID), lambda i: (i, 0)),
        out_shape=jax.ShapeDtypeStruct((NH, 2 * HID), jnp.float32),
    )(x, x, W)


def _tc_prescale(degp2, h2):
    def body(d_ref, h_ref, dinv_ref, hs_ref):
        dv = lax.rsqrt(d_ref[0] + d_ref[1] + 1.0)
        dinv_ref[...] = dv
        hs_ref[...] = h_ref[...] * dv

    return pl.pallas_call(
        body,
        grid=(GRID,),
        in_specs=[
            pl.BlockSpec((NCORE, RB, 2 * HID), lambda i: (0, i, 0)),
            pl.BlockSpec((RB, 2 * HID), lambda i: (i, 0)),
        ],
        out_specs=[
            pl.BlockSpec((RB, 2 * HID), lambda i: (i, 0)),
            pl.BlockSpec((RB, 2 * HID), lambda i: (i, 0)),
        ],
        out_shape=[
            jax.ShapeDtypeStruct((NH, 2 * HID), jnp.float32),
            jax.ShapeDtypeStruct((NH, 2 * HID), jnp.float32),
        ],
    )(degp2, h2)


def _post_math(S_ref, hs_ref, dinv_ref, b_ref, g_ref, bt_ref, LL_ref,
               lb_ref, P_ref):
    t = S_ref[0] + S_ref[1] + hs_ref[...]
    h = jnp.maximum(t * dinv_ref[...] + b_ref[...], 0.0)
    mu = jnp.dot(h, P_ref[...], preferred_element_type=jnp.float32)
    xc = h - mu
    var = jnp.dot(xc * xc, P_ref[...], preferred_element_type=jnp.float32)
    hn = xc * lax.rsqrt(var + 1e-5) * g_ref[...] + bt_ref[...]
    h2 = jnp.dot(hn, LL_ref[...], preferred_element_type=jnp.float32)
    return jnp.maximum(h2 + lb_ref[...], 0.0)


def _tc_post(S2, hs2, dinv2, b2, g2, bt2, LL, lb2, WW, P):
    dd = LL.shape[1]

    def body(S_ref, hs_ref, dinv_ref, b_ref, g_ref, bt_ref, LL_ref, lb_ref,
             ww_ref, p_ref, o_ref):
        h2 = _post_math(S_ref, hs_ref, dinv_ref, b_ref, g_ref, bt_ref,
                        LL_ref, lb_ref, p_ref)
        o_ref[...] = jnp.dot(h2, ww_ref[...],
                             preferred_element_type=jnp.float32) * dinv_ref[...]

    return pl.pallas_call(
        body,
        grid=(GRID,),
        in_specs=[
            pl.BlockSpec((NCORE, RB, 2 * HID), lambda i: (0, i, 0)),
            pl.BlockSpec((RB, 2 * HID), lambda i: (i, 0)),
            pl.BlockSpec((RB, 2 * HID), lambda i: (i, 0)),
            pl.BlockSpec((1, 2 * HID), lambda i: (0, 0)),
            pl.BlockSpec((1, 2 * HID), lambda i: (0, 0)),
            pl.BlockSpec((1, 2 * HID), lambda i: (0, 0)),
            pl.BlockSpec((2 * HID, dd), lambda i: (0, 0)),
            pl.BlockSpec((1, dd), lambda i: (0, 0)),
            pl.BlockSpec((dd, 2 * HID), lambda i: (0, 0)),
            pl.BlockSpec((2 * HID, 2 * HID), lambda i: (0, 0)),
        ],
        out_specs=pl.BlockSpec((RB, 2 * HID), lambda i: (i, 0)),
        out_shape=jax.ShapeDtypeStruct((NH, 2 * HID), jnp.float32),
    )(S2, hs2, dinv2, b2, g2, bt2, LL, lb2, WW, P)


def _tc_final(S2, hs2, dinv2, b2, g2, bt2, LL, lb2, P):
    dd = LL.shape[1]
    d2 = dd // 2

    def body(S_ref, hs_ref, dinv_ref, b_ref, g_ref, bt_ref, LL_ref, lb_ref,
             p_ref, o_ref):
        h2 = _post_math(S_ref, hs_ref, dinv_ref, b_ref, g_ref, bt_ref,
                        LL_ref, lb_ref, p_ref)
        m2 = jnp.max(h2, axis=0, keepdims=True)
        m = jnp.maximum(m2[:, :d2], m2[:, d2:])
        i = pl.program_id(0)

        @pl.when(i == 0)
        def _():
            o_ref[...] = m

        @pl.when(i > 0)
        def _():
            o_ref[...] = jnp.maximum(o_ref[...], m)

    return pl.pallas_call(
        body,
        grid=(GRID,),
        in_specs=[
            pl.BlockSpec((NCORE, RB, 2 * HID), lambda i: (0, i, 0)),
            pl.BlockSpec((RB, 2 * HID), lambda i: (i, 0)),
            pl.BlockSpec((RB, 2 * HID), lambda i: (i, 0)),
            pl.BlockSpec((1, 2 * HID), lambda i: (0, 0)),
            pl.BlockSpec((1, 2 * HID), lambda i: (0, 0)),
            pl.BlockSpec((1, 2 * HID), lambda i: (0, 0)),
            pl.BlockSpec((2 * HID, dd), lambda i: (0, 0)),
            pl.BlockSpec((1, dd), lambda i: (0, 0)),
            pl.BlockSpec((2 * HID, 2 * HID), lambda i: (0, 0)),
        ],
        out_specs=pl.BlockSpec((1, d2), lambda i: (0, 0)),
        out_shape=jax.ShapeDtypeStruct((1, d2), jnp.float32),
    )(S2, hs2, dinv2, b2, g2, bt2, LL, lb2, P)


def kernel(x, edge_index,
           W1, b1, g1, bt1, L1, lb1,
           W2, b2, g2, bt2, L2, lb2,
           W3, b3, g3, bt3, L3, lb3):
    epi = (edge_index % NH) * 2 + edge_index // NH
    edge3 = epi.reshape(2, NW, NCH, C)

    z_deg = jnp.zeros((SLICE, DW), jnp.float32)
    z_acc = jnp.zeros((SLICE, HID), jnp.float32)
    ones_u = jnp.ones((C, DW), jnp.float32)

    def dup(v):
        return jnp.concatenate([v, v]).reshape(1, -1)

    def bdiag(M):
        a, b = M.shape
        Z = jnp.zeros((a, b), M.dtype)
        return jnp.concatenate(
            [jnp.concatenate([M, Z], axis=1),
             jnp.concatenate([Z, M], axis=1)], axis=0)

    P = bdiag(jnp.full((HID, HID), 1.0 / HID, jnp.float32))
    pack2 = lambda a: a.reshape(NCORE, NPAD // 2, 2 * HID)
    unpack = lambda a: a.reshape(N, HID)

    degp = _sc_degree(edge3, ones_u, z_deg)
    h2 = _tc_mm(x, W1)
    degp2 = degp.reshape(NCORE, NPAD // 2, 2 * HID)
    dinv2, hs2 = _tc_prescale(degp2, h2)

    S = pack2(_sc_scatter(unpack(hs2), edge3, z_acc))
    hs2 = _tc_post(S, hs2, dinv2, dup(b1), dup(g1), dup(bt1), bdiag(L1),
                   dup(lb1), bdiag(W2), P)

    S = pack2(_sc_scatter(unpack(hs2), edge3, z_acc))
    hs2 = _tc_post(S, hs2, dinv2, dup(b2), dup(g2), dup(bt2), bdiag(L2),
                   dup(lb2), bdiag(W3), P)

    S = pack2(_sc_scatter(unpack(hs2), edge3, z_acc))
    out = _tc_final(S, hs2, dinv2, dup(b3), dup(g3), dup(bt3), bdiag(L3),
                    dup(lb3), P)
    return out.reshape(L3.shape[1])


# speedup vs baseline: 1.4418x; 1.0541x over previous
import functools

import jax
import jax.numpy as jnp
from jax import lax
from jax.experimental import pallas as pl
from jax.experimental.pallas import tpu as pltpu
from jax.experimental.pallas import tpu_sc as plsc

N = 10000
NH = N // 2
E = 320000
HID = 64
NCORE = 2
NSUB = 16
NW = NCORE * NSUB
NPAD = 10240
SLICE = NPAD // NSUB
C = 125
EPW = E // NW
NCH = EPW // C
DW = 16
RB = 1000
GRID = NH // RB

_mesh = plsc.VectorSubcoreMesh(core_axis_name="c", subcore_axis_name="s")
_sc_params = pltpu.CompilerParams(use_tc_tiling_on_sc=False)


def _sc_degree(edge3, ones_u, zrows):
    @functools.partial(
        pl.kernel,
        out_type=jax.ShapeDtypeStruct((NCORE, NPAD, HID // DW, DW), jnp.float32),
        mesh=_mesh,
        compiler_params=_sc_params,
        scratch_types=[
            pltpu.VMEM((NCH, C), jnp.int32),
            pltpu.VMEM((C, DW), jnp.float32),
            pltpu.VMEM((SLICE, DW), jnp.float32),
            pltpu.VMEM((SLICE, HID // DW, DW), jnp.float32),
            pltpu.SemaphoreType.DMA,
            pltpu.VMEM_SHARED((NPAD, DW), jnp.float32),
        ],
    )
    def k(edge_hbm, ones_hbm, z_hbm, out_hbm, didx, ones_v, nar, wide, sem,
          acc):
        c = lax.axis_index("c")
        s = lax.axis_index("s")
        w = c * NSUB + s
        pltpu.sync_copy(z_hbm, acc.at[pl.ds(s * SLICE, SLICE)])
        pltpu.sync_copy(edge_hbm.at[1, w], didx)
        pltpu.sync_copy(ones_hbm, ones_v)
        plsc.subcore_barrier()

        @pl.loop(0, NCH)
        def _(j):
            pltpu.async_copy(ones_v, acc.at[didx.at[j]], sem, add=True)

        @pl.loop(0, NCH)
        def _(j):
            pltpu.make_async_copy(ones_v, acc.at[didx.at[j]], sem).wait()

        plsc.subcore_barrier()
        pltpu.sync_copy(acc.at[pl.ds(s * SLICE, SLICE)], nar)

        @pl.loop(0, SLICE)
        def _(r):
            v = nar[r]
            for q in range(HID // DW):
                wide[r, q] = v

        pltpu.sync_copy(wide, out_hbm.at[c, pl.ds(s * SLICE, SLICE)])

    return k(edge3, ones_u, zrows)


def _sc_scatter(hs_pk, edge3, zrows):
    @functools.partial(
        pl.kernel,
        out_type=jax.ShapeDtypeStruct((NCORE, NPAD, HID), jnp.float32),
        mesh=_mesh,
        compiler_params=_sc_params,
        scratch_types=[
            pltpu.VMEM((NCH, C), jnp.int32),
            pltpu.VMEM((NCH, C), jnp.int32),
            [pltpu.VMEM((C, HID), jnp.float32)] * 8,
            [pltpu.SemaphoreType.DMA] * 8,
            [pltpu.SemaphoreType.DMA] * 8,
            pltpu.VMEM_SHARED((NPAD, HID), jnp.float32),
        ],
    )
    def k(hs_hbm, edge_hbm, z_hbm, out_hbm, sidx, didx,
          rows, gsem, ssem, acc):
        c = lax.axis_index("c")
        s = lax.axis_index("s")
        w = c * NSUB + s
        pltpu.sync_copy(z_hbm, acc.at[pl.ds(s * SLICE, SLICE)])
        pltpu.sync_copy(edge_hbm.at[0, w], sidx)
        pltpu.sync_copy(edge_hbm.at[1, w], didx)
        plsc.subcore_barrier()

        def fire_g(j, q):
            pltpu.async_copy(hs_hbm.at[sidx.at[j]], rows[q], gsem[q])

        def drain_g(j, q):
            pltpu.make_async_copy(hs_hbm.at[sidx.at[j]], rows[q],
                                  gsem[q]).wait()

        def fire_s(j, q):
            pltpu.async_copy(rows[q], acc.at[didx.at[j]], ssem[q], add=True)

        def drain_s(j, q):
            pltpu.make_async_copy(rows[q], acc.at[didx.at[j]],
                                  ssem[q]).wait()

        for q in range(8):
            fire_g(q, q)

        @pl.loop(0, NCH, step=8)
        def _(j):
            for q in range(8):
                drain_g(j + q, q)
                fire_s(j + q, q)

            @pl.when(j + 8 < NCH)
            def _():
                for q in range(8):
                    drain_s(j + q, q)
                    fire_g(j + 8 + q, q)

        for q in range(8):
            drain_s(NCH - 8 + q, q)

        plsc.subcore_barrier()
        pltpu.sync_copy(acc.at[pl.ds(s * SLICE, SLICE)],
                        out_hbm.at[c, pl.ds(s * SLICE, SLICE)])

    return k(hs_pk, edge3, zrows)


def _tc_mm(x, W):
    d = x.shape[1]

    def body(xa_ref, xb_ref, w_ref, o_ref):
        ha = jnp.dot(xa_ref[...], w_ref[...], preferred_element_type=jnp.float32)
        hb = jnp.dot(xb_ref[...], w_ref[...], preferred_element_type=jnp.float32)
        o_ref[...] = jnp.concatenate([ha, hb], axis=1)

    return pl.pallas_call(
        body,
        grid=(GRID,),
        in_specs=[
            pl.BlockSpec((RB, d), lambda i: (i, 0)),
            pl.BlockSpec((RB, d), lambda i: (i + GRID, 0)),
            pl.BlockSpec((d, HID), lambda i: (0, 0)),
        ],
        out_specs=pl.BlockSpec((RB, 2 * HID), lambda i: (i, 0)),
        out_shape=jax.ShapeDtypeStruct((NH, 2 * HID), jnp.float32),
    )(x, x, W)


def _tc_prescale(degp2, h2):
    def body(d_ref, h_ref, dinv_ref, hs_ref):
        dv = lax.rsqrt(d_ref[0] + d_ref[1] + 1.0)
        dinv_ref[...] = dv
        hs_ref[...] = h_ref[...] * dv

    return pl.pallas_call(
        body,
        grid=(GRID,),
        in_specs=[
            pl.BlockSpec((NCORE, RB, 2 * HID), lambda i: (0, i, 0)),
            pl.BlockSpec((RB, 2 * HID), lambda i: (i, 0)),
        ],
        out_specs=[
            pl.BlockSpec((RB, 2 * HID), lambda i: (i, 0)),
            pl.BlockSpec((RB, 2 * HID), lambda i: (i, 0)),
        ],
        out_shape=[
            jax.ShapeDtypeStruct((NH, 2 * HID), jnp.float32),
            jax.ShapeDtypeStruct((NH, 2 * HID), jnp.float32),
        ],
    )(degp2, h2)


def _post_math(S_ref, hs_ref, dinv_ref, b_ref, g_ref, bt_ref, LL_ref,
               lb_ref, P_ref):
    t = S_ref[0] + S_ref[1] + hs_ref[...]
    h = jnp.maximum(t * dinv_ref[...] + b_ref[...], 0.0)
    mu = jnp.dot(h, P_ref[...], preferred_element_type=jnp.float32)
    xc = h - mu
    var = jnp.dot(xc * xc, P_ref[...], preferred_element_type=jnp.float32)
    hn = xc * lax.rsqrt(var + 1e-5) * g_ref[...] + bt_ref[...]
    h2 = jnp.dot(hn, LL_ref[...], preferred_element_type=jnp.float32)
    return jnp.maximum(h2 + lb_ref[...], 0.0)


def _tc_post(S2, hs2, dinv2, b2, g2, bt2, LL, lb2, WW, P):
    dd = LL.shape[1]

    def body(S_ref, hs_ref, dinv_ref, b_ref, g_ref, bt_ref, LL_ref, lb_ref,
             ww_ref, p_ref, o_ref):
        h2 = _post_math(S_ref, hs_ref, dinv_ref, b_ref, g_ref, bt_ref,
                        LL_ref, lb_ref, p_ref)
        o_ref[...] = jnp.dot(h2, ww_ref[...],
                             preferred_element_type=jnp.float32) * dinv_ref[...]

    return pl.pallas_call(
        body,
        grid=(GRID,),
        in_specs=[
            pl.BlockSpec((NCORE, RB, 2 * HID), lambda i: (0, i, 0)),
            pl.BlockSpec((RB, 2 * HID), lambda i: (i, 0)),
            pl.BlockSpec((RB, 2 * HID), lambda i: (i, 0)),
            pl.BlockSpec((1, 2 * HID), lambda i: (0, 0)),
            pl.BlockSpec((1, 2 * HID), lambda i: (0, 0)),
            pl.BlockSpec((1, 2 * HID), lambda i: (0, 0)),
            pl.BlockSpec((2 * HID, dd), lambda i: (0, 0)),
            pl.BlockSpec((1, dd), lambda i: (0, 0)),
            pl.BlockSpec((dd, 2 * HID), lambda i: (0, 0)),
            pl.BlockSpec((2 * HID, 2 * HID), lambda i: (0, 0)),
        ],
        out_specs=pl.BlockSpec((RB, 2 * HID), lambda i: (i, 0)),
        out_shape=jax.ShapeDtypeStruct((NH, 2 * HID), jnp.float32),
    )(S2, hs2, dinv2, b2, g2, bt2, LL, lb2, WW, P)


def _tc_final(S2, hs2, dinv2, b2, g2, bt2, LL, lb2, P):
    dd = LL.shape[1]
    d2 = dd // 2

    def body(S_ref, hs_ref, dinv_ref, b_ref, g_ref, bt_ref, LL_ref, lb_ref,
             p_ref, o_ref):
        h2 = _post_math(S_ref, hs_ref, dinv_ref, b_ref, g_ref, bt_ref,
                        LL_ref, lb_ref, p_ref)
        m2 = jnp.max(h2, axis=0, keepdims=True)
        m = jnp.maximum(m2[:, :d2], m2[:, d2:])
        i = pl.program_id(0)

        @pl.when(i == 0)
        def _():
            o_ref[...] = m

        @pl.when(i > 0)
        def _():
            o_ref[...] = jnp.maximum(o_ref[...], m)

    return pl.pallas_call(
        body,
        grid=(GRID,),
        in_specs=[
            pl.BlockSpec((NCORE, RB, 2 * HID), lambda i: (0, i, 0)),
            pl.BlockSpec((RB, 2 * HID), lambda i: (i, 0)),
            pl.BlockSpec((RB, 2 * HID), lambda i: (i, 0)),
            pl.BlockSpec((1, 2 * HID), lambda i: (0, 0)),
            pl.BlockSpec((1, 2 * HID), lambda i: (0, 0)),
            pl.BlockSpec((1, 2 * HID), lambda i: (0, 0)),
            pl.BlockSpec((2 * HID, dd), lambda i: (0, 0)),
            pl.BlockSpec((1, dd), lambda i: (0, 0)),
            pl.BlockSpec((2 * HID, 2 * HID), lambda i: (0, 0)),
        ],
        out_specs=pl.BlockSpec((1, d2), lambda i: (0, 0)),
        out_shape=jax.ShapeDtypeStruct((1, d2), jnp.float32),
    )(S2, hs2, dinv2, b2, g2, bt2, LL, lb2, P)


def kernel(x, edge_index,
           W1, b1, g1, bt1, L1, lb1,
           W2, b2, g2, bt2, L2, lb2,
           W3, b3, g3, bt3, L3, lb3):
    epi = (edge_index % NH) * 2 + edge_index // NH
    edge3 = epi.reshape(2, NW, NCH, C)

    z_deg = jnp.zeros((SLICE, DW), jnp.float32)
    z_acc = jnp.zeros((SLICE, HID), jnp.float32)
    ones_u = jnp.ones((C, DW), jnp.float32)

    def dup(v):
        return jnp.concatenate([v, v]).reshape(1, -1)

    def bdiag(M):
        a, b = M.shape
        Z = jnp.zeros((a, b), M.dtype)
        return jnp.concatenate(
            [jnp.concatenate([M, Z], axis=1),
             jnp.concatenate([Z, M], axis=1)], axis=0)

    P = bdiag(jnp.full((HID, HID), 1.0 / HID, jnp.float32))
    pack2 = lambda a: a.reshape(NCORE, NPAD // 2, 2 * HID)
    unpack = lambda a: a.reshape(N, HID)

    degp = _sc_degree(edge3, ones_u, z_deg)
    h2 = _tc_mm(x, W1)
    degp2 = degp.reshape(NCORE, NPAD // 2, 2 * HID)
    dinv2, hs2 = _tc_prescale(degp2, h2)

    S = pack2(_sc_scatter(unpack(hs2), edge3, z_acc))
    hs2 = _tc_post(S, hs2, dinv2, dup(b1), dup(g1), dup(bt1), bdiag(L1),
                   dup(lb1), bdiag(W2), P)

    S = pack2(_sc_scatter(unpack(hs2), edge3, z_acc))
    hs2 = _tc_post(S, hs2, dinv2, dup(b2), dup(g2), dup(bt2), bdiag(L2),
                   dup(lb2), bdiag(W3), P)

    S = pack2(_sc_scatter(unpack(hs2), edge3, z_acc))
    out = _tc_final(S, hs2, dinv2, dup(b3), dup(g3), dup(bt3), bdiag(L3),
                    dup(lb3), P)
    return out.reshape(L3.shape[1])
